# baseline (speedup 1.0000x reference)
import jax
import jax.numpy as jnp
from jax.experimental import pallas as pl
from jax.experimental.pallas import tpu as pltpu

_B, _S, _D = 1, 2048, 768
_E, _D1, _D2 = 8, 768, 768
_K = 2
_BS = 256


def _gating_kernel(x_ref, wa_ref, ba_ref, wg_ref, bg_ref, idx_ref, wts_ref):
    x = x_ref[...]
    scores = jnp.dot(x, wa_ref[...],
                     preferred_element_type=jnp.float32) + ba_ref[0, 0]
    m = jnp.max(scores)
    p = jnp.exp(scores - m)
    aw = p / jnp.sum(p)
    pooled = jnp.sum(x * aw, axis=0, keepdims=True)
    logits = jnp.dot(pooled, wg_ref[...],
                     preferred_element_type=jnp.float32) + bg_ref[...]
    gm = jnp.max(logits)
    ge = jnp.exp(logits - gm)
    gate = ge / jnp.sum(ge)
    ii = jax.lax.broadcasted_iota(jnp.int32, (1, _E), 1)
    v1 = jnp.max(gate)
    i1 = jnp.min(jnp.where(gate == v1, ii, _E))
    masked = jnp.where(ii == i1, -1.0, gate)
    v2 = jnp.max(masked)
    i2 = jnp.min(jnp.where(masked == v2, ii, _E))
    denom = v1 + v2 + 1e-9
    idx_ref[0] = i1
    idx_ref[1] = i2
    wts_ref[0] = v1 / denom
    wts_ref[1] = v2 / denom


def _gelu(v):
    return 0.5 * v * (1.0 + jax.lax.erf(v * 0.7071067811865476))


def _expert_kernel(idx_ref, x_ref, w1a_ref, b1a_ref, w2a_ref, b2a_ref,
                   w1b_ref, b1b_ref, w2b_ref, b2b_ref, wts_ref, out_ref):
    x = x_ref[...]
    h = _gelu(jnp.dot(x, w1a_ref[0], preferred_element_type=jnp.float32)
              + b1a_ref[0])
    o = _gelu(jnp.dot(h, w2a_ref[0], preferred_element_type=jnp.float32)
              + b2a_ref[0])
    acc = wts_ref[0] * o
    h = _gelu(jnp.dot(x, w1b_ref[0], preferred_element_type=jnp.float32)
              + b1b_ref[0])
    o = _gelu(jnp.dot(h, w2b_ref[0], preferred_element_type=jnp.float32)
              + b2b_ref[0])
    out_ref[...] = acc + wts_ref[1] * o


def kernel(x, Wa, ba, Wg, bg, W1, b1, W2, b2):
    x2 = x.reshape(_S, _D)
    idx, wts = pl.pallas_call(
        _gating_kernel,
        out_shape=(
            jax.ShapeDtypeStruct((_K,), jnp.int32),
            jax.ShapeDtypeStruct((_K,), jnp.float32),
        ),
        in_specs=[
            pl.BlockSpec(memory_space=pltpu.VMEM),
            pl.BlockSpec(memory_space=pltpu.VMEM),
            pl.BlockSpec(memory_space=pltpu.SMEM),
            pl.BlockSpec(memory_space=pltpu.VMEM),
            pl.BlockSpec(memory_space=pltpu.VMEM),
        ],
        out_specs=(
            pl.BlockSpec(memory_space=pltpu.SMEM),
            pl.BlockSpec(memory_space=pltpu.SMEM),
        ),
    )(x2, Wa, ba.reshape(1, 1), Wg, bg.reshape(1, _E))

    b1r = b1.reshape(_E, 1, _D1)
    b2r = b2.reshape(_E, 1, _D2)
    grid_spec = pltpu.PrefetchScalarGridSpec(
        num_scalar_prefetch=1,
        grid=(_S // _BS,),
        in_specs=[
            pl.BlockSpec((_BS, _D), lambda i, idx: (i, 0)),
            pl.BlockSpec((1, _D, _D1), lambda i, idx: (idx[0], 0, 0)),
            pl.BlockSpec((1, 1, _D1), lambda i, idx: (idx[0], 0, 0)),
            pl.BlockSpec((1, _D1, _D2), lambda i, idx: (idx[0], 0, 0)),
            pl.BlockSpec((1, 1, _D2), lambda i, idx: (idx[0], 0, 0)),
            pl.BlockSpec((1, _D, _D1), lambda i, idx: (idx[1], 0, 0)),
            pl.BlockSpec((1, 1, _D1), lambda i, idx: (idx[1], 0, 0)),
            pl.BlockSpec((1, _D1, _D2), lambda i, idx: (idx[1], 0, 0)),
            pl.BlockSpec((1, 1, _D2), lambda i, idx: (idx[1], 0, 0)),
            pl.BlockSpec(memory_space=pltpu.SMEM),
        ],
        out_specs=pl.BlockSpec((_BS, _D2), lambda i, idx: (i, 0)),
    )
    out = pl.pallas_call(
        _expert_kernel,
        grid_spec=grid_spec,
        out_shape=jax.ShapeDtypeStruct((_S, _D2), jnp.float32),
    )(idx, x2, W1, b1r, W2, b2r, W1, b1r, W2, b2r, wts)
    return out.reshape(_B, _S, _D2)


# speedup vs baseline: 1.0253x; 1.0253x over previous
import jax
import jax.numpy as jnp
from jax.experimental import pallas as pl
from jax.experimental.pallas import tpu as pltpu

_B, _S, _D = 1, 2048, 768
_E, _D1, _D2 = 8, 768, 768
_K = 2
_BS = 1024


def _gating_kernel(x_ref, wa_ref, ba_ref, wg_ref, bg_ref, idx_ref, wts_ref):
    x = x_ref[...]
    scores = jnp.dot(x, wa_ref[...],
                     preferred_element_type=jnp.float32) + ba_ref[0, 0]
    m = jnp.max(scores)
    p = jnp.exp(scores - m)
    aw = p / jnp.sum(p)
    pooled = jnp.sum(x * aw, axis=0, keepdims=True)
    logits = jnp.dot(pooled, wg_ref[...],
                     preferred_element_type=jnp.float32) + bg_ref[...]
    gm = jnp.max(logits)
    ge = jnp.exp(logits - gm)
    gate = ge / jnp.sum(ge)
    ii = jax.lax.broadcasted_iota(jnp.int32, (1, _E), 1)
    v1 = jnp.max(gate)
    i1 = jnp.min(jnp.where(gate == v1, ii, _E))
    masked = jnp.where(ii == i1, -1.0, gate)
    v2 = jnp.max(masked)
    i2 = jnp.min(jnp.where(masked == v2, ii, _E))
    denom = v1 + v2 + 1e-9
    idx_ref[0] = i1
    idx_ref[1] = i2
    wts_ref[0] = v1 / denom
    wts_ref[1] = v2 / denom


def _gelu(v):
    return 0.5 * v * (1.0 + jax.lax.erf(v * 0.7071067811865476))


def _expert_kernel(idx_ref, x_ref, w1a_ref, b1a_ref, w2a_ref, b2a_ref,
                   w1b_ref, b1b_ref, w2b_ref, b2b_ref, wts_ref, out_ref):
    x = x_ref[...]
    h = _gelu(jnp.dot(x, w1a_ref[0], preferred_element_type=jnp.float32)
              + b1a_ref[0])
    o = _gelu(jnp.dot(h, w2a_ref[0], preferred_element_type=jnp.float32)
              + b2a_ref[0])
    acc = wts_ref[0] * o
    h = _gelu(jnp.dot(x, w1b_ref[0], preferred_element_type=jnp.float32)
              + b1b_ref[0])
    o = _gelu(jnp.dot(h, w2b_ref[0], preferred_element_type=jnp.float32)
              + b2b_ref[0])
    out_ref[...] = acc + wts_ref[1] * o


def kernel(x, Wa, ba, Wg, bg, W1, b1, W2, b2):
    x2 = x.reshape(_S, _D)
    idx, wts = pl.pallas_call(
        _gating_kernel,
        out_shape=(
            jax.ShapeDtypeStruct((_K,), jnp.int32),
            jax.ShapeDtypeStruct((_K,), jnp.float32),
        ),
        in_specs=[
            pl.BlockSpec(memory_space=pltpu.VMEM),
            pl.BlockSpec(memory_space=pltpu.VMEM),
            pl.BlockSpec(memory_space=pltpu.SMEM),
            pl.BlockSpec(memory_space=pltpu.VMEM),
            pl.BlockSpec(memory_space=pltpu.VMEM),
        ],
        out_specs=(
            pl.BlockSpec(memory_space=pltpu.SMEM),
            pl.BlockSpec(memory_space=pltpu.SMEM),
        ),
    )(x2, Wa, ba.reshape(1, 1), Wg, bg.reshape(1, _E))

    b1r = b1.reshape(_E, 1, _D1)
    b2r = b2.reshape(_E, 1, _D2)
    grid_spec = pltpu.PrefetchScalarGridSpec(
        num_scalar_prefetch=1,
        grid=(_S // _BS,),
        in_specs=[
            pl.BlockSpec((_BS, _D), lambda i, idx: (i, 0)),
            pl.BlockSpec((1, _D, _D1), lambda i, idx: (idx[0], 0, 0)),
            pl.BlockSpec((1, 1, _D1), lambda i, idx: (idx[0], 0, 0)),
            pl.BlockSpec((1, _D1, _D2), lambda i, idx: (idx[0], 0, 0)),
            pl.BlockSpec((1, 1, _D2), lambda i, idx: (idx[0], 0, 0)),
            pl.BlockSpec((1, _D, _D1), lambda i, idx: (idx[1], 0, 0)),
            pl.BlockSpec((1, 1, _D1), lambda i, idx: (idx[1], 0, 0)),
            pl.BlockSpec((1, _D1, _D2), lambda i, idx: (idx[1], 0, 0)),
            pl.BlockSpec((1, 1, _D2), lambda i, idx: (idx[1], 0, 0)),
            pl.BlockSpec(memory_space=pltpu.SMEM),
        ],
        out_specs=pl.BlockSpec((_BS, _D2), lambda i, idx: (i, 0)),
    )
    out = pl.pallas_call(
        _expert_kernel,
        grid_spec=grid_spec,
        out_shape=jax.ShapeDtypeStruct((_S, _D2), jnp.float32),
    )(idx, x2, W1, b1r, W2, b2r, W1, b1r, W2, b2r, wts)
    return out.reshape(_B, _S, _D2)


# speedup vs baseline: 1.0438x; 1.0180x over previous
import jax
import jax.numpy as jnp
from jax.experimental import pallas as pl
from jax.experimental.pallas import tpu as pltpu

_B, _S, _D = 1, 2048, 768
_E, _D1, _D2 = 8, 768, 768
_K = 2
_BS = 512


def _gating_kernel(x_ref, wa_ref, ba_ref, wg_ref, bg_ref, idx_ref, wts_ref):
    x = x_ref[...]
    scores = jnp.dot(x, wa_ref[...],
                     preferred_element_type=jnp.float32) + ba_ref[0, 0]
    m = jnp.max(scores)
    p = jnp.exp(scores - m)
    aw = p / jnp.sum(p)
    pooled = jnp.sum(x * aw, axis=0, keepdims=True)
    logits = jnp.dot(pooled, wg_ref[...],
                     preferred_element_type=jnp.float32) + bg_ref[...]
    gm = jnp.max(logits)
    ge = jnp.exp(logits - gm)
    gate = ge / jnp.sum(ge)
    ii = jax.lax.broadcasted_iota(jnp.int32, (1, _E), 1)
    v1 = jnp.max(gate)
    i1 = jnp.min(jnp.where(gate == v1, ii, _E))
    masked = jnp.where(ii == i1, -1.0, gate)
    v2 = jnp.max(masked)
    i2 = jnp.min(jnp.where(masked == v2, ii, _E))
    denom = v1 + v2 + 1e-9
    idx_ref[0] = i1
    idx_ref[1] = i2
    wts_ref[0] = v1 / denom
    wts_ref[1] = v2 / denom


def _gelu(v):
    return 0.5 * v * (1.0 + jax.lax.erf(v * 0.7071067811865476))


def _expert_kernel(idx_ref, x_ref, w1a_ref, b1a_ref, w2a_ref, b2a_ref,
                   w1b_ref, b1b_ref, w2b_ref, b2b_ref, wts_ref, out_ref):
    x = x_ref[...].astype(jnp.bfloat16)
    h = _gelu(jnp.dot(x, w1a_ref[0].astype(jnp.bfloat16),
                      preferred_element_type=jnp.float32) + b1a_ref[0])
    o = _gelu(jnp.dot(h.astype(jnp.bfloat16), w2a_ref[0].astype(jnp.bfloat16),
                      preferred_element_type=jnp.float32) + b2a_ref[0])
    acc = wts_ref[0] * o
    h = _gelu(jnp.dot(x, w1b_ref[0].astype(jnp.bfloat16),
                      preferred_element_type=jnp.float32) + b1b_ref[0])
    o = _gelu(jnp.dot(h.astype(jnp.bfloat16), w2b_ref[0].astype(jnp.bfloat16),
                      preferred_element_type=jnp.float32) + b2b_ref[0])
    out_ref[...] = acc + wts_ref[1] * o


def kernel(x, Wa, ba, Wg, bg, W1, b1, W2, b2):
    x2 = x.reshape(_S, _D)
    idx, wts = pl.pallas_call(
        _gating_kernel,
        out_shape=(
            jax.ShapeDtypeStruct((_K,), jnp.int32),
            jax.ShapeDtypeStruct((_K,), jnp.float32),
        ),
        in_specs=[
            pl.BlockSpec(memory_space=pltpu.VMEM),
            pl.BlockSpec(memory_space=pltpu.VMEM),
            pl.BlockSpec(memory_space=pltpu.SMEM),
            pl.BlockSpec(memory_space=pltpu.VMEM),
            pl.BlockSpec(memory_space=pltpu.VMEM),
        ],
        out_specs=(
            pl.BlockSpec(memory_space=pltpu.SMEM),
            pl.BlockSpec(memory_space=pltpu.SMEM),
        ),
    )(x2, Wa, ba.reshape(1, 1), Wg, bg.reshape(1, _E))

    b1r = b1.reshape(_E, 1, _D1)
    b2r = b2.reshape(_E, 1, _D2)
    grid_spec = pltpu.PrefetchScalarGridSpec(
        num_scalar_prefetch=1,
        grid=(_S // _BS,),
        in_specs=[
            pl.BlockSpec((_BS, _D), lambda i, idx: (i, 0)),
            pl.BlockSpec((1, _D, _D1), lambda i, idx: (idx[0], 0, 0)),
            pl.BlockSpec((1, 1, _D1), lambda i, idx: (idx[0], 0, 0)),
            pl.BlockSpec((1, _D1, _D2), lambda i, idx: (idx[0], 0, 0)),
            pl.BlockSpec((1, 1, _D2), lambda i, idx: (idx[0], 0, 0)),
            pl.BlockSpec((1, _D, _D1), lambda i, idx: (idx[1], 0, 0)),
            pl.BlockSpec((1, 1, _D1), lambda i, idx: (idx[1], 0, 0)),
            pl.BlockSpec((1, _D1, _D2), lambda i, idx: (idx[1], 0, 0)),
            pl.BlockSpec((1, 1, _D2), lambda i, idx: (idx[1], 0, 0)),
            pl.BlockSpec(memory_space=pltpu.SMEM),
        ],
        out_specs=pl.BlockSpec((_BS, _D2), lambda i, idx: (i, 0)),
    )
    out = pl.pallas_call(
        _expert_kernel,
        grid_spec=grid_spec,
        out_shape=jax.ShapeDtypeStruct((_S, _D2), jnp.float32),
    )(idx, x2, W1, b1r, W2, b2r, W1, b1r, W2, b2r, wts)
    return out.reshape(_B, _S, _D2)
